# baseline (speedup 1.0000x reference)
import jax
import jax.numpy as jnp
from jax import lax
from jax.experimental import pallas as pl
from jax.experimental.pallas import tpu as pltpu
from jax.experimental.pallas import tpu_sc as plsc

NG = 50000
NCN = 50000
NGO = 10000
E = 150000
F = 128
CH = 32
NCHUNK = 4
NSUB = 16
GRP = 128
GPW = 80
EROWS = NSUB * GPW
EPAD = EROWS * GRP
GPH = GPW // 2
UNR = 4
ITH = GPH // UNR


def _acc_geom(n_dst):
    if n_dst == 50000:
        return 50176, 3136, 2960
    if n_dst == 10000:
        return 10240, 640, 400
    raise ValueError(n_dst)


def _conv_block(ztab, srcx, dstp, zeros, out, n_dst,
                src_v, dst_v, rows, acc, gsem, ssem, c, s):
    ACC, STRIPE, LAST = _acc_geom(n_dst)
    for p in range(2):
        ck = 2 * p + c
        pltpu.sync_copy(zeros.at[pl.ds(0, STRIPE)],
                        acc.at[pl.ds(s * STRIPE, STRIPE)])
        plsc.subcore_barrier()
        for half in range(2):
            pltpu.sync_copy(srcx.at[ck, s, half], src_v)
            pltpu.sync_copy(dstp.at[s, half], dst_v)

            def step(t, carry):
                base = t * UNR
                gds = [pltpu.async_copy(ztab.at[src_v.at[base + b]],
                                        rows.at[b], gsem.at[b])
                       for b in range(UNR)]
                sds = []
                for b in range(UNR):
                    gds[b].wait()
                    sds.append(pltpu.async_copy(rows.at[b],
                                                acc.at[dst_v.at[base + b]],
                                                ssem.at[b], add=True))
                for d in sds:
                    d.wait()
                return carry

            lax.fori_loop(0, ITH, step, 0)
        plsc.subcore_barrier()

        @pl.when(s < NSUB - 1)
        def _():
            pltpu.sync_copy(acc.at[pl.ds(s * STRIPE, STRIPE)],
                            out.at[pl.ds(s * STRIPE, STRIPE), ck])

        @pl.when(s == NSUB - 1)
        def _():
            pltpu.sync_copy(acc.at[pl.ds((NSUB - 1) * STRIPE, LAST)],
                            out.at[pl.ds((NSUB - 1) * STRIPE, LAST), ck])

        plsc.subcore_barrier()


def _make_agg_batch(n_dsts):
    k = len(n_dsts)
    mesh = plsc.VectorSubcoreMesh(core_axis_name="c", subcore_axis_name="s")

    def body(*refs):
        ztabs = refs[0:k]
        srcxs = refs[k:2 * k]
        dstps = refs[2 * k:3 * k]
        zeros = refs[3 * k]
        outs = refs[3 * k + 1:4 * k + 1]
        src_v, dst_v, rows, acc, gsem, ssem = refs[4 * k + 1:]
        c = lax.axis_index("c")
        s = lax.axis_index("s")
        for i in range(k):
            _conv_block(ztabs[i], srcxs[i], dstps[i], zeros, outs[i],
                        n_dsts[i], src_v, dst_v, rows, acc, gsem, ssem,
                        c, s)

    return pl.kernel(
        body,
        out_type=[jax.ShapeDtypeStruct((n, NCHUNK, CH), jnp.float32)
                  for n in n_dsts],
        mesh=mesh,
        scratch_types=[
            pltpu.VMEM((GPH, GRP), jnp.int32),
            pltpu.VMEM((GPH, GRP), jnp.int32),
            pltpu.VMEM((UNR, GRP, CH), jnp.float32),
            pltpu.VMEM_SHARED((max(_acc_geom(n)[0] for n in n_dsts), CH),
                              jnp.float32),
            pltpu.SemaphoreType.DMA((UNR,)),
            pltpu.SemaphoreType.DMA((UNR,)),
        ],
        compiler_params=pltpu.CompilerParams(use_tc_tiling_on_sc=False),
    )


_HSIZES = [NG, NCN, NCN, NG, NG, NGO, NGO, NCN]
DW = 16


def _make_hist():
    mesh = plsc.VectorSubcoreMesh(core_axis_name="c", subcore_axis_name="s")
    out_types = [jax.ShapeDtypeStruct((n, DW), jnp.float32)
                 for n in _HSIZES]

    def body(*refs):
        idx_in = refs[0:8]
        ones_h, zeros16 = refs[8], refs[9]
        outs = refs[10:18]
        dst_v, ones_v, acc, hsem = refs[18:22]
        c = lax.axis_index("c")
        s = lax.axis_index("s")
        pltpu.sync_copy(ones_h, ones_v)
        for half in range(2):
            @pl.when(c == half)
            def _(half=half):
                for h in range(4 * half, 4 * half + 4):
                    n = _HSIZES[h]
                    _, STRIPE, LAST = _acc_geom(n)
                    pltpu.sync_copy(zeros16.at[pl.ds(0, STRIPE)],
                                    acc.at[pl.ds(s * STRIPE, STRIPE)])
                    pltpu.sync_copy(idx_in[h].at[s], dst_v)
                    plsc.subcore_barrier()

                    for hh in range(2):
                        def grp(t, carry, hh=hh):
                            base = t * UNR
                            sds = [pltpu.async_copy(
                                       ones_v,
                                       acc.at[dst_v.at[hh, base + b]],
                                       hsem.at[b], add=True)
                                   for b in range(UNR)]
                            for d in sds:
                                d.wait()
                            return carry

                        lax.fori_loop(0, ITH, grp, 0)
                    plsc.subcore_barrier()

                    @pl.when(s < NSUB - 1)
                    def _():
                        pltpu.sync_copy(
                            acc.at[pl.ds(s * STRIPE, STRIPE)],
                            outs[h].at[pl.ds(s * STRIPE, STRIPE)])

                    @pl.when(s == NSUB - 1)
                    def _():
                        pltpu.sync_copy(
                            acc.at[pl.ds((NSUB - 1) * STRIPE, LAST)],
                            outs[h].at[pl.ds((NSUB - 1) * STRIPE, LAST)])

                    plsc.subcore_barrier()

    return pl.kernel(
        body,
        out_type=out_types,
        mesh=mesh,
        scratch_types=[
            pltpu.VMEM((2, GPH, GRP), jnp.int32),
            pltpu.VMEM((GRP, DW), jnp.float32),
            pltpu.VMEM_SHARED((50176, DW), jnp.float32),
            pltpu.SemaphoreType.DMA((UNR,)),
        ],
        compiler_params=pltpu.CompilerParams(use_tc_tiling_on_sc=False),
    )


def _prep_edges(e, n_src, n_dst):
    src = e[0]
    dst = e[1]
    pad = EPAD - E
    src_h = jnp.concatenate(
        [src, jnp.full((pad,), n_src, jnp.int32)]).reshape(
            NSUB, 2, GPH, GRP)
    dst_h = jnp.concatenate(
        [dst, jnp.full((pad,), n_dst, jnp.int32)]).reshape(
            NSUB, 2, GPH, GRP)
    src0 = jnp.concatenate([src, jnp.zeros((pad,), jnp.int32)])
    srcx = (src0[None, :] * NCHUNK
            + jnp.arange(NCHUNK, dtype=jnp.int32)[:, None]).reshape(
                NCHUNK, NSUB, 2, GPH, GRP)
    return src_h, dst_h, srcx, dst_h


_B = 2000


def _nrm(d):
    return lax.rsqrt(jnp.maximum(d, 1.0))


def _scale_call(x, deg):
    n = x.shape[0]

    def body(x_ref, d_ref, o_ref):
        o_ref[...] = x_ref[...] * _nrm(d_ref[...])

    return pl.pallas_call(
        body,
        grid=(n // _B,),
        in_specs=[pl.BlockSpec((_B, F), lambda i: (i, 0)),
                  pl.BlockSpec((_B, 1), lambda i: (i, 0))],
        out_specs=pl.BlockSpec((_B, F), lambda i: (i, 0)),
        out_shape=jax.ShapeDtypeStruct((n, F), jnp.float32),
    )(x, deg)


def _assemble(a_ref, d_ref):
    return a_ref[...] * _nrm(d_ref[...])


def _post1(agg, dd, W, b, ns_list):
    n = agg.shape[0]
    m = len(ns_list)

    def body(a_ref, d_ref, w_ref, b_ref, *rest):
        ns_refs = rest[:m]
        outs = rest[m:]
        xb = _assemble(a_ref, d_ref)
        y = jnp.maximum(
            jnp.dot(xb, w_ref[...], preferred_element_type=jnp.float32)
            + b_ref[...], 0.0)
        for j in range(m):
            outs[j][...] = y * _nrm(ns_refs[j][...])

    return pl.pallas_call(
        body,
        grid=(n // _B,),
        in_specs=[pl.BlockSpec((_B, F), lambda i: (i, 0)),
                  pl.BlockSpec((_B, 1), lambda i: (i, 0)),
                  pl.BlockSpec((F, F), lambda i: (0, 0)),
                  pl.BlockSpec((1, F), lambda i: (0, 0))]
        + [pl.BlockSpec((_B, 1), lambda i: (i, 0))] * m,
        out_specs=[pl.BlockSpec((_B, F), lambda i: (i, 0))] * m,
        out_shape=[jax.ShapeDtypeStruct((n, F), jnp.float32)] * m,
    )(agg, dd, W, b.reshape(1, F), *ns_list)


def _post2(agg1, dd1, Wa, ba, agg2, dd2, Wb, bb, ns=None, final=None):
    n = agg1.shape[0]
    m = len(ns) if ns is not None else 0

    def body(a1, d1, wa, ba_, a2, d2, wb, bb_, *rest):
        y = (jnp.maximum(
                jnp.dot(_assemble(a1, d1), wa[...],
                        preferred_element_type=jnp.float32) + ba_[...], 0.0)
             + jnp.maximum(
                jnp.dot(_assemble(a2, d2), wb[...],
                        preferred_element_type=jnp.float32) + bb_[...], 0.0))
        if final is not None:
            wd, bd_, out = rest
            out[...] = (jnp.dot(y, wd[...],
                                preferred_element_type=jnp.float32)
                        + bd_[...])
        else:
            ns_refs = rest[:m]
            outs = rest[m:]
            for j in range(m):
                outs[j][...] = y * _nrm(ns_refs[j][...])

    base_specs = [pl.BlockSpec((_B, F), lambda i: (i, 0)),
                  pl.BlockSpec((_B, 1), lambda i: (i, 0)),
                  pl.BlockSpec((F, F), lambda i: (0, 0)),
                  pl.BlockSpec((1, F), lambda i: (0, 0))]
    in_specs = base_specs + base_specs
    args = [agg1, dd1, Wa, ba.reshape(1, F), agg2, dd2, Wb, bb.reshape(1, F)]
    if final is not None:
        wd, bd_ = final
        in_specs += [pl.BlockSpec((F, F), lambda i: (0, 0)),
                     pl.BlockSpec((1, F), lambda i: (0, 0))]
        args += [wd, bd_.reshape(1, F)]
        out_specs = pl.BlockSpec((_B, F), lambda i: (i, 0))
        out_shape = jax.ShapeDtypeStruct((n, F), jnp.float32)
    else:
        in_specs += [pl.BlockSpec((_B, 1), lambda i: (i, 0))] * m
        args += list(ns)
        out_specs = [pl.BlockSpec((_B, F), lambda i: (i, 0))] * m
        out_shape = [jax.ShapeDtypeStruct((n, F), jnp.float32)] * m
    return pl.pallas_call(
        body,
        grid=(n // _B,),
        in_specs=in_specs,
        out_specs=out_specs,
        out_shape=out_shape,
    )(*args)


def kernel(x_gene, x_cell, x_gotem, edges_gene2cell, edges_cell2gene,
           edges_gene2gotem, edges_gotem2cell, Weg, beg, Wgo, bgo,
           W1, b1, W2, b2, W3, b3, Wd, bd):
    zeros = jnp.zeros((3136, CH), jnp.float32)
    ones_h = jnp.ones((GRP, DW), jnp.float32)
    zeros16 = jnp.zeros((3136, DW), jnp.float32)

    rels = [(edges_gene2cell, NG, NCN),
            (edges_cell2gene, NCN, NG),
            (edges_gene2gotem, NG, NGO),
            (edges_gotem2cell, NGO, NCN)]
    prep = [_prep_edges(e, ns_, nd_) for e, ns_, nd_ in rels]

    hist_in = []
    for (src_h, dst_h, _, _) in prep:
        hist_in += [src_h, dst_h]
    degs = _make_hist()(*hist_in, ones_h, zeros16)

    def _deg(i, n):
        return degs[i][:, :1]

    s0 = _deg(0, NG)
    d0 = _deg(1, NCN)
    s1 = _deg(2, NCN)
    d1 = _deg(3, NG)
    s2 = _deg(4, NG)
    d2 = _deg(5, NGO)
    s3 = _deg(6, NGO)
    d3 = _deg(7, NCN)

    def BATCH(convs):
        n_dsts = tuple(rels[ri][2] for _, ri in convs)
        f = _make_agg_batch(n_dsts)
        ztabs = [z.reshape(-1, CH) for z, _ in convs]
        srcxs = [prep[ri][2] for _, ri in convs]
        dstps = [prep[ri][3] for _, ri in convs]
        outs = f(*ztabs, *srcxs, *dstps, zeros)
        return [o.reshape(n, F) for o, n in zip(outs, n_dsts)]

    zx_g = _scale_call(x_gene, s0)
    zx_c = _scale_call(x_cell, s1)
    zx_go = _scale_call(x_gotem, s3)

    A_c2g, A_g2c, A_go2c = BATCH([(zx_c, 1), (zx_g, 0), (zx_go, 3)])
    g_s0, g_s2 = _post1(A_c2g, d1, Weg[1], beg[1], [s0, s2])
    c_s1, = _post2(A_g2c, d0, Weg[0], beg[0],
                   A_go2c, d3, Weg[3], beg[3], ns=[s1])

    A_g2go, A4, A6 = BATCH([(g_s2, 2), (g_s0, 0), (c_s1, 1)])
    go_s3, = _post1(A_g2go, d2, Wgo, bgo, [s3])
    hg1_s2, = _post1(A6, d1, W1[1], b1[1], [s2])

    A5, A8 = BATCH([(go_s3, 3), (hg1_s2, 2)])
    hc1_s1, = _post2(A4, d0, W1[0], b1[0], A5, d3, W1[3], b1[3], ns=[s1])
    hgo2_s3, = _post1(A8, d2, W2[2], b2[2], [s3])

    A7, A10 = BATCH([(hc1_s1, 1), (hgo2_s3, 3)])
    hg2_s0, = _post1(A7, d1, W2[1], b2[1], [s0])

    A9, = BATCH([(hg2_s0, 0)])
    out = _post2(A9, d0, W3[0], b3[0], A10, d3, W3[3], b3[3],
                 final=(Wd[1], bd[1]))
    return out


# speedup vs baseline: 1.5199x; 1.5199x over previous
import jax
import jax.numpy as jnp
from jax import lax
from jax.experimental import pallas as pl
from jax.experimental.pallas import tpu as pltpu
from jax.experimental.pallas import tpu_sc as plsc

NG = 50000
NCN = 50000
NGO = 10000
E = 150000
F = 128
CH = 32
NCHUNK = 4
NSUB = 16
GRP = 128
GPW = 76
EROWS = NSUB * GPW
EPAD = EROWS * GRP
GPH = GPW // 2


def _acc_geom(n_dst):
    if n_dst == 50000:
        return 50176, 3136, 2960
    if n_dst == 10000:
        return 10240, 640, 400
    raise ValueError(n_dst)


def _conv_block(ztab, srcx, dstp, zeros, out, n_dst,
                src_v, dst_v, rows, acc, gsem, c, s):
    ACC, STRIPE, LAST = _acc_geom(n_dst)
    for p in range(2):
        ck = 2 * p + c
        pltpu.sync_copy(zeros.at[pl.ds(0, STRIPE)],
                        acc.at[pl.ds(s * STRIPE, STRIPE)])
        plsc.subcore_barrier()
        for half in range(2):
            pltpu.sync_copy(srcx.at[ck, s, half], src_v)
            pltpu.sync_copy(dstp.at[s, half], dst_v)

            def step(g, carry):
                pltpu.async_copy(ztab.at[src_v.at[g]], rows, gsem).wait()
                pltpu.sync_copy(rows, acc.at[dst_v.at[g]], add=True)
                return carry

            lax.fori_loop(0, GPH, step, 0)
        plsc.subcore_barrier()

        @pl.when(s < NSUB - 1)
        def _():
            pltpu.sync_copy(acc.at[pl.ds(s * STRIPE, STRIPE)],
                            out.at[pl.ds(s * STRIPE, STRIPE), ck])

        @pl.when(s == NSUB - 1)
        def _():
            pltpu.sync_copy(acc.at[pl.ds((NSUB - 1) * STRIPE, LAST)],
                            out.at[pl.ds((NSUB - 1) * STRIPE, LAST), ck])

        plsc.subcore_barrier()


def _make_agg_batch(n_dsts):
    k = len(n_dsts)
    mesh = plsc.VectorSubcoreMesh(core_axis_name="c", subcore_axis_name="s")

    def body(*refs):
        ztabs = refs[0:k]
        srcxs = refs[k:2 * k]
        dstps = refs[2 * k:3 * k]
        zeros = refs[3 * k]
        outs = refs[3 * k + 1:4 * k + 1]
        src_v, dst_v, rows, acc, gsem = refs[4 * k + 1:]
        c = lax.axis_index("c")
        s = lax.axis_index("s")
        for i in range(k):
            _conv_block(ztabs[i], srcxs[i], dstps[i], zeros, outs[i],
                        n_dsts[i], src_v, dst_v, rows, acc, gsem, c, s)

    return pl.kernel(
        body,
        out_type=[jax.ShapeDtypeStruct((n, NCHUNK, CH), jnp.float32)
                  for n in n_dsts],
        mesh=mesh,
        scratch_types=[
            pltpu.VMEM((GPH, GRP), jnp.int32),
            pltpu.VMEM((GPH, GRP), jnp.int32),
            pltpu.VMEM((GRP, CH), jnp.float32),
            pltpu.VMEM_SHARED((max(_acc_geom(n)[0] for n in n_dsts), CH),
                              jnp.float32),
            pltpu.SemaphoreType.DMA,
        ],
        compiler_params=pltpu.CompilerParams(use_tc_tiling_on_sc=False),
    )


_HSIZES = [NG, NCN, NCN, NG, NG, NGO, NGO, NCN]
DW = 16


def _make_hist():
    mesh = plsc.VectorSubcoreMesh(core_axis_name="c", subcore_axis_name="s")
    out_types = [jax.ShapeDtypeStruct((n, DW), jnp.float32)
                 for n in _HSIZES]

    def body(*refs):
        idx_in = refs[0:8]
        ones_h, zeros16 = refs[8], refs[9]
        outs = refs[10:18]
        dst_v, ones_v, acc, hsem = refs[18:22]
        c = lax.axis_index("c")
        s = lax.axis_index("s")
        pltpu.sync_copy(ones_h, ones_v)
        for half in range(2):
            @pl.when(c == half)
            def _(half=half):
                for h in range(4 * half, 4 * half + 4):
                    n = _HSIZES[h]
                    _, STRIPE, LAST = _acc_geom(n)
                    pltpu.sync_copy(zeros16.at[pl.ds(0, STRIPE)],
                                    acc.at[pl.ds(s * STRIPE, STRIPE)])
                    pltpu.sync_copy(idx_in[h].at[s], dst_v)
                    plsc.subcore_barrier()

                    for hh in range(2):
                        def grp(g, carry, hh=hh):
                            pltpu.sync_copy(ones_v,
                                            acc.at[dst_v.at[hh, g]],
                                            add=True)
                            return carry

                        lax.fori_loop(0, GPH, grp, 0)
                    plsc.subcore_barrier()

                    @pl.when(s < NSUB - 1)
                    def _():
                        pltpu.sync_copy(
                            acc.at[pl.ds(s * STRIPE, STRIPE)],
                            outs[h].at[pl.ds(s * STRIPE, STRIPE)])

                    @pl.when(s == NSUB - 1)
                    def _():
                        pltpu.sync_copy(
                            acc.at[pl.ds((NSUB - 1) * STRIPE, LAST)],
                            outs[h].at[pl.ds((NSUB - 1) * STRIPE, LAST)])

                    plsc.subcore_barrier()

    return pl.kernel(
        body,
        out_type=out_types,
        mesh=mesh,
        scratch_types=[
            pltpu.VMEM((2, GPH, GRP), jnp.int32),
            pltpu.VMEM((GRP, DW), jnp.float32),
            pltpu.VMEM_SHARED((50176, DW), jnp.float32),
            pltpu.SemaphoreType.DMA,
        ],
        compiler_params=pltpu.CompilerParams(use_tc_tiling_on_sc=False),
    )


def _prep_edges(e, n_src, n_dst):
    src = e[0]
    dst = e[1]
    pad = EPAD - E
    src_h = jnp.concatenate(
        [src, jnp.full((pad,), n_src, jnp.int32)]).reshape(
            NSUB, 2, GPH, GRP)
    dst_h = jnp.concatenate(
        [dst, jnp.full((pad,), n_dst, jnp.int32)]).reshape(
            NSUB, 2, GPH, GRP)
    src0 = jnp.concatenate([src, jnp.zeros((pad,), jnp.int32)])
    srcx = (src0[None, :] * NCHUNK
            + jnp.arange(NCHUNK, dtype=jnp.int32)[:, None]).reshape(
                NCHUNK, NSUB, 2, GPH, GRP)
    return src_h, dst_h, srcx, dst_h


_B = 2000


def _nrm(d):
    return lax.rsqrt(jnp.maximum(d, 1.0))


def _scale_call(x, deg):
    n = x.shape[0]

    def body(x_ref, d_ref, o_ref):
        o_ref[...] = x_ref[...] * _nrm(d_ref[...])

    return pl.pallas_call(
        body,
        grid=(n // _B,),
        in_specs=[pl.BlockSpec((_B, F), lambda i: (i, 0)),
                  pl.BlockSpec((_B, 1), lambda i: (i, 0))],
        out_specs=pl.BlockSpec((_B, F), lambda i: (i, 0)),
        out_shape=jax.ShapeDtypeStruct((n, F), jnp.float32),
    )(x, deg)


def _assemble(a_ref, d_ref):
    return a_ref[...] * _nrm(d_ref[...])


def _post1(agg, dd, W, b, ns_list):
    n = agg.shape[0]
    m = len(ns_list)

    def body(a_ref, d_ref, w_ref, b_ref, *rest):
        ns_refs = rest[:m]
        outs = rest[m:]
        xb = _assemble(a_ref, d_ref)
        y = jnp.maximum(
            jnp.dot(xb, w_ref[...], preferred_element_type=jnp.float32)
            + b_ref[...], 0.0)
        for j in range(m):
            outs[j][...] = y * _nrm(ns_refs[j][...])

    return pl.pallas_call(
        body,
        grid=(n // _B,),
        in_specs=[pl.BlockSpec((_B, F), lambda i: (i, 0)),
                  pl.BlockSpec((_B, 1), lambda i: (i, 0)),
                  pl.BlockSpec((F, F), lambda i: (0, 0)),
                  pl.BlockSpec((1, F), lambda i: (0, 0))]
        + [pl.BlockSpec((_B, 1), lambda i: (i, 0))] * m,
        out_specs=[pl.BlockSpec((_B, F), lambda i: (i, 0))] * m,
        out_shape=[jax.ShapeDtypeStruct((n, F), jnp.float32)] * m,
    )(agg, dd, W, b.reshape(1, F), *ns_list)


def _post2(agg1, dd1, Wa, ba, agg2, dd2, Wb, bb, ns=None, final=None):
    n = agg1.shape[0]
    m = len(ns) if ns is not None else 0

    def body(a1, d1, wa, ba_, a2, d2, wb, bb_, *rest):
        y = (jnp.maximum(
                jnp.dot(_assemble(a1, d1), wa[...],
                        preferred_element_type=jnp.float32) + ba_[...], 0.0)
             + jnp.maximum(
                jnp.dot(_assemble(a2, d2), wb[...],
                        preferred_element_type=jnp.float32) + bb_[...], 0.0))
        if final is not None:
            wd, bd_, out = rest
            out[...] = (jnp.dot(y, wd[...],
                                preferred_element_type=jnp.float32)
                        + bd_[...])
        else:
            ns_refs = rest[:m]
            outs = rest[m:]
            for j in range(m):
                outs[j][...] = y * _nrm(ns_refs[j][...])

    base_specs = [pl.BlockSpec((_B, F), lambda i: (i, 0)),
                  pl.BlockSpec((_B, 1), lambda i: (i, 0)),
                  pl.BlockSpec((F, F), lambda i: (0, 0)),
                  pl.BlockSpec((1, F), lambda i: (0, 0))]
    in_specs = base_specs + base_specs
    args = [agg1, dd1, Wa, ba.reshape(1, F), agg2, dd2, Wb, bb.reshape(1, F)]
    if final is not None:
        wd, bd_ = final
        in_specs += [pl.BlockSpec((F, F), lambda i: (0, 0)),
                     pl.BlockSpec((1, F), lambda i: (0, 0))]
        args += [wd, bd_.reshape(1, F)]
        out_specs = pl.BlockSpec((_B, F), lambda i: (i, 0))
        out_shape = jax.ShapeDtypeStruct((n, F), jnp.float32)
    else:
        in_specs += [pl.BlockSpec((_B, 1), lambda i: (i, 0))] * m
        args += list(ns)
        out_specs = [pl.BlockSpec((_B, F), lambda i: (i, 0))] * m
        out_shape = [jax.ShapeDtypeStruct((n, F), jnp.float32)] * m
    return pl.pallas_call(
        body,
        grid=(n // _B,),
        in_specs=in_specs,
        out_specs=out_specs,
        out_shape=out_shape,
    )(*args)


def kernel(x_gene, x_cell, x_gotem, edges_gene2cell, edges_cell2gene,
           edges_gene2gotem, edges_gotem2cell, Weg, beg, Wgo, bgo,
           W1, b1, W2, b2, W3, b3, Wd, bd):
    zeros = jnp.zeros((3136, CH), jnp.float32)
    ones_h = jnp.ones((GRP, DW), jnp.float32)
    zeros16 = jnp.zeros((3136, DW), jnp.float32)

    rels = [(edges_gene2cell, NG, NCN),
            (edges_cell2gene, NCN, NG),
            (edges_gene2gotem, NG, NGO),
            (edges_gotem2cell, NGO, NCN)]
    prep = [_prep_edges(e, ns_, nd_) for e, ns_, nd_ in rels]

    hist_in = []
    for (src_h, dst_h, _, _) in prep:
        hist_in += [src_h, dst_h]
    degs = _make_hist()(*hist_in, ones_h, zeros16)

    def _deg(i, n):
        return degs[i][:, :1]

    s0 = _deg(0, NG)
    d0 = _deg(1, NCN)
    s1 = _deg(2, NCN)
    d1 = _deg(3, NG)
    s2 = _deg(4, NG)
    d2 = _deg(5, NGO)
    s3 = _deg(6, NGO)
    d3 = _deg(7, NCN)

    def BATCH(convs):
        n_dsts = tuple(rels[ri][2] for _, ri in convs)
        f = _make_agg_batch(n_dsts)
        ztabs = [z.reshape(-1, CH) for z, _ in convs]
        srcxs = [prep[ri][2] for _, ri in convs]
        dstps = [prep[ri][3] for _, ri in convs]
        outs = f(*ztabs, *srcxs, *dstps, zeros)
        return [o.reshape(n, F) for o, n in zip(outs, n_dsts)]

    zx_g = _scale_call(x_gene, s0)
    zx_c = _scale_call(x_cell, s1)
    zx_go = _scale_call(x_gotem, s3)

    A_c2g, A_g2c, A_go2c = BATCH([(zx_c, 1), (zx_g, 0), (zx_go, 3)])
    g_s0, g_s2 = _post1(A_c2g, d1, Weg[1], beg[1], [s0, s2])
    c_s1, = _post2(A_g2c, d0, Weg[0], beg[0],
                   A_go2c, d3, Weg[3], beg[3], ns=[s1])

    A_g2go, A4, A6 = BATCH([(g_s2, 2), (g_s0, 0), (c_s1, 1)])
    go_s3, = _post1(A_g2go, d2, Wgo, bgo, [s3])
    hg1_s2, = _post1(A6, d1, W1[1], b1[1], [s2])

    A5, A8 = BATCH([(go_s3, 3), (hg1_s2, 2)])
    hc1_s1, = _post2(A4, d0, W1[0], b1[0], A5, d3, W1[3], b1[3], ns=[s1])
    hgo2_s3, = _post1(A8, d2, W2[2], b2[2], [s3])

    A7, A10 = BATCH([(hc1_s1, 1), (hgo2_s3, 3)])
    hg2_s0, = _post1(A7, d1, W2[1], b2[1], [s0])

    A9, = BATCH([(hg2_s0, 0)])
    out = _post2(A9, d0, W3[0], b3[0], A10, d3, W3[3], b3[3],
                 final=(Wd[1], bd[1]))
    return out


# speedup vs baseline: 1.7005x; 1.1188x over previous
import jax
import jax.numpy as jnp
from jax import lax
from jax.experimental import pallas as pl
from jax.experimental.pallas import tpu as pltpu
from jax.experimental.pallas import tpu_sc as plsc

NG = 50000
NCN = 50000
NGO = 10000
E = 150000
F = 128
CH = 32
NCHUNK = 4
NSUB = 16
GRP = 128
GPW = 76
EROWS = NSUB * GPW
EPAD = EROWS * GRP
GPH = GPW // 2
OPE = 256
OPH = GPH * GRP // OPE


def _acc_geom(n_dst):
    if n_dst == 50000:
        return 50176, 3136, 2960
    if n_dst == 10000:
        return 10240, 640, 400
    raise ValueError(n_dst)


def _conv_block(ztab, srcx, dstp, zeros, out, n_dst,
                src_v, dst_v, rows, acc, gsem, c, s):
    ACC, STRIPE, LAST = _acc_geom(n_dst)
    for p in range(2):
        ck = 2 * p + c
        pltpu.sync_copy(zeros.at[pl.ds(0, STRIPE)],
                        acc.at[pl.ds(s * STRIPE, STRIPE)])
        plsc.subcore_barrier()
        for half in range(2):
            pltpu.sync_copy(srcx.at[ck, s, half], src_v)
            pltpu.sync_copy(dstp.at[s, half], dst_v)

            def step(g, carry):
                pltpu.async_copy(ztab.at[src_v.at[g]], rows, gsem).wait()
                pltpu.sync_copy(rows, acc.at[dst_v.at[g]], add=True)
                return carry

            lax.fori_loop(0, OPH, step, 0)
        plsc.subcore_barrier()

        @pl.when(s < NSUB - 1)
        def _():
            pltpu.sync_copy(acc.at[pl.ds(s * STRIPE, STRIPE)],
                            out.at[pl.ds(s * STRIPE, STRIPE), ck])

        @pl.when(s == NSUB - 1)
        def _():
            pltpu.sync_copy(acc.at[pl.ds((NSUB - 1) * STRIPE, LAST)],
                            out.at[pl.ds((NSUB - 1) * STRIPE, LAST), ck])

        plsc.subcore_barrier()


def _make_agg_batch(n_dsts):
    k = len(n_dsts)
    mesh = plsc.VectorSubcoreMesh(core_axis_name="c", subcore_axis_name="s")

    def body(*refs):
        ztabs = refs[0:k]
        srcxs = refs[k:2 * k]
        dstps = refs[2 * k:3 * k]
        zeros = refs[3 * k]
        outs = refs[3 * k + 1:4 * k + 1]
        src_v, dst_v, rows, acc, gsem = refs[4 * k + 1:]
        c = lax.axis_index("c")
        s = lax.axis_index("s")
        for i in range(k):
            _conv_block(ztabs[i], srcxs[i], dstps[i], zeros, outs[i],
                        n_dsts[i], src_v, dst_v, rows, acc, gsem, c, s)

    return pl.kernel(
        body,
        out_type=[jax.ShapeDtypeStruct((n, NCHUNK, CH), jnp.float32)
                  for n in n_dsts],
        mesh=mesh,
        scratch_types=[
            pltpu.VMEM((OPH, OPE), jnp.int32),
            pltpu.VMEM((OPH, OPE), jnp.int32),
            pltpu.VMEM((OPE, CH), jnp.float32),
            pltpu.VMEM_SHARED((max(_acc_geom(n)[0] for n in n_dsts), CH),
                              jnp.float32),
            pltpu.SemaphoreType.DMA,
        ],
        compiler_params=pltpu.CompilerParams(use_tc_tiling_on_sc=False),
    )


_HSIZES = [NG, NCN, NCN, NG, NG, NGO, NGO, NCN]
DW = 16


def _make_hist():
    mesh = plsc.VectorSubcoreMesh(core_axis_name="c", subcore_axis_name="s")
    out_types = [jax.ShapeDtypeStruct((n, DW), jnp.float32)
                 for n in _HSIZES]

    def body(*refs):
        idx_in = refs[0:8]
        ones_h, zeros16 = refs[8], refs[9]
        outs = refs[10:18]
        dst_v, ones_v, acc, hsem = refs[18:22]
        c = lax.axis_index("c")
        s = lax.axis_index("s")
        pltpu.sync_copy(ones_h, ones_v)
        for half in range(2):
            @pl.when(c == half)
            def _(half=half):
                for h in range(4 * half, 4 * half + 4):
                    n = _HSIZES[h]
                    _, STRIPE, LAST = _acc_geom(n)
                    pltpu.sync_copy(zeros16.at[pl.ds(0, STRIPE)],
                                    acc.at[pl.ds(s * STRIPE, STRIPE)])
                    pltpu.sync_copy(idx_in[h].at[s], dst_v)
                    plsc.subcore_barrier()

                    for hh in range(2):
                        def grp(g, carry, hh=hh):
                            pltpu.sync_copy(ones_v,
                                            acc.at[dst_v.at[hh, g]],
                                            add=True)
                            return carry

                        lax.fori_loop(0, OPH, grp, 0)
                    plsc.subcore_barrier()

                    @pl.when(s < NSUB - 1)
                    def _():
                        pltpu.sync_copy(
                            acc.at[pl.ds(s * STRIPE, STRIPE)],
                            outs[h].at[pl.ds(s * STRIPE, STRIPE)])

                    @pl.when(s == NSUB - 1)
                    def _():
                        pltpu.sync_copy(
                            acc.at[pl.ds((NSUB - 1) * STRIPE, LAST)],
                            outs[h].at[pl.ds((NSUB - 1) * STRIPE, LAST)])

                    plsc.subcore_barrier()

    return pl.kernel(
        body,
        out_type=out_types,
        mesh=mesh,
        scratch_types=[
            pltpu.VMEM((2, OPH, OPE), jnp.int32),
            pltpu.VMEM((OPE, DW), jnp.float32),
            pltpu.VMEM_SHARED((50176, DW), jnp.float32),
            pltpu.SemaphoreType.DMA,
        ],
        compiler_params=pltpu.CompilerParams(use_tc_tiling_on_sc=False),
    )


def _prep_edges(e, n_src, n_dst):
    src = e[0]
    dst = e[1]
    pad = EPAD - E
    src_h = jnp.concatenate(
        [src, jnp.full((pad,), n_src, jnp.int32)]).reshape(
            NSUB, 2, OPH, OPE)
    dst_h = jnp.concatenate(
        [dst, jnp.full((pad,), n_dst, jnp.int32)]).reshape(
            NSUB, 2, OPH, OPE)
    src0 = jnp.concatenate([src, jnp.zeros((pad,), jnp.int32)])
    srcx = (src0[None, :] * NCHUNK
            + jnp.arange(NCHUNK, dtype=jnp.int32)[:, None]).reshape(
                NCHUNK, NSUB, 2, OPH, OPE)
    return src_h, dst_h, srcx, dst_h


_B = 2000


def _nrm(d):
    return lax.rsqrt(jnp.maximum(d, 1.0))


def _scale_call(x, deg):
    n = x.shape[0]

    def body(x_ref, d_ref, o_ref):
        o_ref[...] = x_ref[...] * _nrm(d_ref[...])

    return pl.pallas_call(
        body,
        grid=(n // _B,),
        in_specs=[pl.BlockSpec((_B, F), lambda i: (i, 0)),
                  pl.BlockSpec((_B, 1), lambda i: (i, 0))],
        out_specs=pl.BlockSpec((_B, F), lambda i: (i, 0)),
        out_shape=jax.ShapeDtypeStruct((n, F), jnp.float32),
    )(x, deg)


def _assemble(a_ref, d_ref):
    return a_ref[...] * _nrm(d_ref[...])


def _post1(agg, dd, W, b, ns_list):
    n = agg.shape[0]
    m = len(ns_list)

    def body(a_ref, d_ref, w_ref, b_ref, *rest):
        ns_refs = rest[:m]
        outs = rest[m:]
        xb = _assemble(a_ref, d_ref)
        y = jnp.maximum(
            jnp.dot(xb, w_ref[...], preferred_element_type=jnp.float32)
            + b_ref[...], 0.0)
        for j in range(m):
            outs[j][...] = y * _nrm(ns_refs[j][...])

    return pl.pallas_call(
        body,
        grid=(n // _B,),
        in_specs=[pl.BlockSpec((_B, F), lambda i: (i, 0)),
                  pl.BlockSpec((_B, 1), lambda i: (i, 0)),
                  pl.BlockSpec((F, F), lambda i: (0, 0)),
                  pl.BlockSpec((1, F), lambda i: (0, 0))]
        + [pl.BlockSpec((_B, 1), lambda i: (i, 0))] * m,
        out_specs=[pl.BlockSpec((_B, F), lambda i: (i, 0))] * m,
        out_shape=[jax.ShapeDtypeStruct((n, F), jnp.float32)] * m,
    )(agg, dd, W, b.reshape(1, F), *ns_list)


def _post2(agg1, dd1, Wa, ba, agg2, dd2, Wb, bb, ns=None, final=None):
    n = agg1.shape[0]
    m = len(ns) if ns is not None else 0

    def body(a1, d1, wa, ba_, a2, d2, wb, bb_, *rest):
        y = (jnp.maximum(
                jnp.dot(_assemble(a1, d1), wa[...],
                        preferred_element_type=jnp.float32) + ba_[...], 0.0)
             + jnp.maximum(
                jnp.dot(_assemble(a2, d2), wb[...],
                        preferred_element_type=jnp.float32) + bb_[...], 0.0))
        if final is not None:
            wd, bd_, out = rest
            out[...] = (jnp.dot(y, wd[...],
                                preferred_element_type=jnp.float32)
                        + bd_[...])
        else:
            ns_refs = rest[:m]
            outs = rest[m:]
            for j in range(m):
                outs[j][...] = y * _nrm(ns_refs[j][...])

    base_specs = [pl.BlockSpec((_B, F), lambda i: (i, 0)),
                  pl.BlockSpec((_B, 1), lambda i: (i, 0)),
                  pl.BlockSpec((F, F), lambda i: (0, 0)),
                  pl.BlockSpec((1, F), lambda i: (0, 0))]
    in_specs = base_specs + base_specs
    args = [agg1, dd1, Wa, ba.reshape(1, F), agg2, dd2, Wb, bb.reshape(1, F)]
    if final is not None:
        wd, bd_ = final
        in_specs += [pl.BlockSpec((F, F), lambda i: (0, 0)),
                     pl.BlockSpec((1, F), lambda i: (0, 0))]
        args += [wd, bd_.reshape(1, F)]
        out_specs = pl.BlockSpec((_B, F), lambda i: (i, 0))
        out_shape = jax.ShapeDtypeStruct((n, F), jnp.float32)
    else:
        in_specs += [pl.BlockSpec((_B, 1), lambda i: (i, 0))] * m
        args += list(ns)
        out_specs = [pl.BlockSpec((_B, F), lambda i: (i, 0))] * m
        out_shape = [jax.ShapeDtypeStruct((n, F), jnp.float32)] * m
    return pl.pallas_call(
        body,
        grid=(n // _B,),
        in_specs=in_specs,
        out_specs=out_specs,
        out_shape=out_shape,
    )(*args)


def kernel(x_gene, x_cell, x_gotem, edges_gene2cell, edges_cell2gene,
           edges_gene2gotem, edges_gotem2cell, Weg, beg, Wgo, bgo,
           W1, b1, W2, b2, W3, b3, Wd, bd):
    zeros = jnp.zeros((3136, CH), jnp.float32)
    ones_h = jnp.ones((OPE, DW), jnp.float32)
    zeros16 = jnp.zeros((3136, DW), jnp.float32)

    rels = [(edges_gene2cell, NG, NCN),
            (edges_cell2gene, NCN, NG),
            (edges_gene2gotem, NG, NGO),
            (edges_gotem2cell, NGO, NCN)]
    prep = [_prep_edges(e, ns_, nd_) for e, ns_, nd_ in rels]

    hist_in = []
    for (src_h, dst_h, _, _) in prep:
        hist_in += [src_h, dst_h]
    degs = _make_hist()(*hist_in, ones_h, zeros16)

    def _deg(i, n):
        return degs[i][:, :1]

    s0 = _deg(0, NG)
    d0 = _deg(1, NCN)
    s1 = _deg(2, NCN)
    d1 = _deg(3, NG)
    s2 = _deg(4, NG)
    d2 = _deg(5, NGO)
    s3 = _deg(6, NGO)
    d3 = _deg(7, NCN)

    def BATCH(convs):
        res = []
        for z, ri in convs:
            n_dst = rels[ri][2]
            f = _make_agg_batch((n_dst,))
            out, = f(z.reshape(-1, CH), prep[ri][2], prep[ri][3], zeros)
            res.append(out.reshape(n_dst, F))
        return res

    zx_g = _scale_call(x_gene, s0)
    zx_c = _scale_call(x_cell, s1)
    zx_go = _scale_call(x_gotem, s3)

    A_c2g, A_g2c, A_go2c = BATCH([(zx_c, 1), (zx_g, 0), (zx_go, 3)])
    g_s0, g_s2 = _post1(A_c2g, d1, Weg[1], beg[1], [s0, s2])
    c_s1, = _post2(A_g2c, d0, Weg[0], beg[0],
                   A_go2c, d3, Weg[3], beg[3], ns=[s1])

    A_g2go, A4, A6 = BATCH([(g_s2, 2), (g_s0, 0), (c_s1, 1)])
    go_s3, = _post1(A_g2go, d2, Wgo, bgo, [s3])
    hg1_s2, = _post1(A6, d1, W1[1], b1[1], [s2])

    A5, A8 = BATCH([(go_s3, 3), (hg1_s2, 2)])
    hc1_s1, = _post2(A4, d0, W1[0], b1[0], A5, d3, W1[3], b1[3], ns=[s1])
    hgo2_s3, = _post1(A8, d2, W2[2], b2[2], [s3])

    A7, A10 = BATCH([(hc1_s1, 1), (hgo2_s3, 3)])
    hg2_s0, = _post1(A7, d1, W2[1], b2[1], [s0])

    A9, = BATCH([(hg2_s0, 0)])
    out = _post2(A9, d0, W3[0], b3[0], A10, d3, W3[3], b3[3],
                 final=(Wd[1], bd[1]))
    return out


# speedup vs baseline: 2.2424x; 1.3187x over previous
import jax
import jax.numpy as jnp
from jax import lax
from jax.experimental import pallas as pl
from jax.experimental.pallas import tpu as pltpu
from jax.experimental.pallas import tpu_sc as plsc

NG = 50000
NCN = 50000
NGO = 10000
E = 150000
F = 128
CH = 32
NCHUNK = 4
NSUB = 16
GRP = 128
GPW = 74
EROWS = NSUB * GPW
EPAD = EROWS * GRP


def _acc_geom(n_dst):
    if n_dst == 50000:
        return 50176, 3136, 2960
    if n_dst == 10000:
        return 10240, 640, 400
    raise ValueError(n_dst)


def _conv_block(ztab, srcx, dstp, zeros, out, n_dst,
                src_v, dst_v, rows, acc, gsem, c, s):
    ACC, STRIPE, LAST = _acc_geom(n_dst)
    pltpu.sync_copy(dstp.at[s], dst_v)
    for p in range(2):
        ck = 2 * p + c
        pltpu.sync_copy(srcx.at[ck, s], src_v)
        pltpu.sync_copy(zeros.at[pl.ds(0, STRIPE)],
                        acc.at[pl.ds(s * STRIPE, STRIPE)])
        plsc.subcore_barrier()

        def step(g, carry):
            pltpu.async_copy(ztab.at[src_v.at[g]], rows, gsem).wait()
            pltpu.sync_copy(rows, acc.at[dst_v.at[g]], add=True)
            return carry

        lax.fori_loop(0, GPW, step, 0)
        plsc.subcore_barrier()

        @pl.when(s < NSUB - 1)
        def _():
            pltpu.sync_copy(acc.at[pl.ds(s * STRIPE, STRIPE)],
                            out.at[pl.ds(s * STRIPE, STRIPE), ck])

        @pl.when(s == NSUB - 1)
        def _():
            pltpu.sync_copy(acc.at[pl.ds((NSUB - 1) * STRIPE, LAST)],
                            out.at[pl.ds((NSUB - 1) * STRIPE, LAST), ck])

        plsc.subcore_barrier()


def _make_agg_batch(n_dsts):
    k = len(n_dsts)
    mesh = plsc.VectorSubcoreMesh(core_axis_name="c", subcore_axis_name="s")

    def body(*refs):
        ztabs = refs[0:k]
        srcxs = refs[k:2 * k]
        dstps = refs[2 * k:3 * k]
        zeros = refs[3 * k]
        outs = refs[3 * k + 1:4 * k + 1]
        src_v, dst_v, rows, acc, gsem = refs[4 * k + 1:]
        c = lax.axis_index("c")
        s = lax.axis_index("s")
        for i in range(k):
            _conv_block(ztabs[i], srcxs[i], dstps[i], zeros, outs[i],
                        n_dsts[i], src_v, dst_v, rows, acc, gsem, c, s)

    return pl.kernel(
        body,
        out_type=[jax.ShapeDtypeStruct((n, NCHUNK, CH), jnp.float32)
                  for n in n_dsts],
        mesh=mesh,
        scratch_types=[
            pltpu.VMEM((GPW, GRP), jnp.int32),
            pltpu.VMEM((GPW, GRP), jnp.int32),
            pltpu.VMEM((GRP, CH), jnp.float32),
            pltpu.VMEM_SHARED((max(_acc_geom(n)[0] for n in n_dsts), CH),
                              jnp.float32),
            pltpu.SemaphoreType.DMA,
        ],
        compiler_params=pltpu.CompilerParams(use_tc_tiling_on_sc=False),
    )


_HSIZES = [NG, NCN, NCN, NG, NG, NGO, NGO, NCN]
DW = 16


def _make_hist():
    mesh = plsc.VectorSubcoreMesh(core_axis_name="c", subcore_axis_name="s")
    out_types = [jax.ShapeDtypeStruct((n, DW), jnp.float32)
                 for n in _HSIZES]

    def body(*refs):
        idx_in = refs[0:8]
        ones_h, zeros16 = refs[8], refs[9]
        outs = refs[10:18]
        dst_v, ones_v, acc, hsem = refs[18:22]
        c = lax.axis_index("c")
        s = lax.axis_index("s")
        pltpu.sync_copy(ones_h, ones_v)
        for half in range(2):
            @pl.when(c == half)
            def _(half=half):
                for h in range(4 * half, 4 * half + 4):
                    n = _HSIZES[h]
                    _, STRIPE, LAST = _acc_geom(n)
                    pltpu.sync_copy(zeros16.at[pl.ds(0, STRIPE)],
                                    acc.at[pl.ds(s * STRIPE, STRIPE)])
                    pltpu.sync_copy(idx_in[h].at[s], dst_v)
                    plsc.subcore_barrier()

                    def grp(g, carry):
                        pltpu.sync_copy(ones_v, acc.at[dst_v.at[g]],
                                        add=True)
                        return carry

                    lax.fori_loop(0, GPW, grp, 0)
                    plsc.subcore_barrier()

                    @pl.when(s < NSUB - 1)
                    def _():
                        pltpu.sync_copy(
                            acc.at[pl.ds(s * STRIPE, STRIPE)],
                            outs[h].at[pl.ds(s * STRIPE, STRIPE)])

                    @pl.when(s == NSUB - 1)
                    def _():
                        pltpu.sync_copy(
                            acc.at[pl.ds((NSUB - 1) * STRIPE, LAST)],
                            outs[h].at[pl.ds((NSUB - 1) * STRIPE, LAST)])

                    plsc.subcore_barrier()

    return pl.kernel(
        body,
        out_type=out_types,
        mesh=mesh,
        scratch_types=[
            pltpu.VMEM((GPW, GRP), jnp.int32),
            pltpu.VMEM((GRP, DW), jnp.float32),
            pltpu.VMEM_SHARED((50176, DW), jnp.float32),
            pltpu.SemaphoreType.DMA,
        ],
        compiler_params=pltpu.CompilerParams(use_tc_tiling_on_sc=False),
    )


def _prep_edges(e, n_src, n_dst):
    src = e[0]
    dst = e[1]
    pad = EPAD - E
    src_h = jnp.concatenate(
        [src, jnp.full((pad,), n_src, jnp.int32)]).reshape(
            NSUB, GPW, GRP)
    dst_h = jnp.concatenate(
        [dst, jnp.full((pad,), n_dst, jnp.int32)]).reshape(
            NSUB, GPW, GRP)
    src0 = jnp.concatenate([src, jnp.zeros((pad,), jnp.int32)])
    srcx = (src0[None, :] * NCHUNK
            + jnp.arange(NCHUNK, dtype=jnp.int32)[:, None]).reshape(
                NCHUNK, NSUB, GPW, GRP)
    return src_h, dst_h, srcx, dst_h


_B = 2000


def _nrm(d):
    return lax.rsqrt(jnp.maximum(d, 1.0))


def _scale_call(x, deg):
    n = x.shape[0]

    def body(x_ref, d_ref, o_ref):
        o_ref[...] = x_ref[...] * _nrm(d_ref[...])

    return pl.pallas_call(
        body,
        grid=(n // _B,),
        in_specs=[pl.BlockSpec((_B, F), lambda i: (i, 0)),
                  pl.BlockSpec((_B, 1), lambda i: (i, 0))],
        out_specs=pl.BlockSpec((_B, F), lambda i: (i, 0)),
        out_shape=jax.ShapeDtypeStruct((n, F), jnp.float32),
    )(x, deg)


def _assemble(a_ref, d_ref):
    return a_ref[...] * _nrm(d_ref[...])


def _post1(agg, dd, W, b, ns_list):
    n = agg.shape[0]
    m = len(ns_list)

    def body(a_ref, d_ref, w_ref, b_ref, *rest):
        ns_refs = rest[:m]
        outs = rest[m:]
        xb = _assemble(a_ref, d_ref)
        y = jnp.maximum(
            jnp.dot(xb, w_ref[...], preferred_element_type=jnp.float32)
            + b_ref[...], 0.0)
        for j in range(m):
            outs[j][...] = y * _nrm(ns_refs[j][...])

    return pl.pallas_call(
        body,
        grid=(n // _B,),
        in_specs=[pl.BlockSpec((_B, F), lambda i: (i, 0)),
                  pl.BlockSpec((_B, 1), lambda i: (i, 0)),
                  pl.BlockSpec((F, F), lambda i: (0, 0)),
                  pl.BlockSpec((1, F), lambda i: (0, 0))]
        + [pl.BlockSpec((_B, 1), lambda i: (i, 0))] * m,
        out_specs=[pl.BlockSpec((_B, F), lambda i: (i, 0))] * m,
        out_shape=[jax.ShapeDtypeStruct((n, F), jnp.float32)] * m,
    )(agg, dd, W, b.reshape(1, F), *ns_list)


def _post2(agg1, dd1, Wa, ba, agg2, dd2, Wb, bb, ns=None, final=None):
    n = agg1.shape[0]
    m = len(ns) if ns is not None else 0

    def body(a1, d1, wa, ba_, a2, d2, wb, bb_, *rest):
        y = (jnp.maximum(
                jnp.dot(_assemble(a1, d1), wa[...],
                        preferred_element_type=jnp.float32) + ba_[...], 0.0)
             + jnp.maximum(
                jnp.dot(_assemble(a2, d2), wb[...],
                        preferred_element_type=jnp.float32) + bb_[...], 0.0))
        if final is not None:
            wd, bd_, out = rest
            out[...] = (jnp.dot(y, wd[...],
                                preferred_element_type=jnp.float32)
                        + bd_[...])
        else:
            ns_refs = rest[:m]
            outs = rest[m:]
            for j in range(m):
                outs[j][...] = y * _nrm(ns_refs[j][...])

    base_specs = [pl.BlockSpec((_B, F), lambda i: (i, 0)),
                  pl.BlockSpec((_B, 1), lambda i: (i, 0)),
                  pl.BlockSpec((F, F), lambda i: (0, 0)),
                  pl.BlockSpec((1, F), lambda i: (0, 0))]
    in_specs = base_specs + base_specs
    args = [agg1, dd1, Wa, ba.reshape(1, F), agg2, dd2, Wb, bb.reshape(1, F)]
    if final is not None:
        wd, bd_ = final
        in_specs += [pl.BlockSpec((F, F), lambda i: (0, 0)),
                     pl.BlockSpec((1, F), lambda i: (0, 0))]
        args += [wd, bd_.reshape(1, F)]
        out_specs = pl.BlockSpec((_B, F), lambda i: (i, 0))
        out_shape = jax.ShapeDtypeStruct((n, F), jnp.float32)
    else:
        in_specs += [pl.BlockSpec((_B, 1), lambda i: (i, 0))] * m
        args += list(ns)
        out_specs = [pl.BlockSpec((_B, F), lambda i: (i, 0))] * m
        out_shape = [jax.ShapeDtypeStruct((n, F), jnp.float32)] * m
    return pl.pallas_call(
        body,
        grid=(n // _B,),
        in_specs=in_specs,
        out_specs=out_specs,
        out_shape=out_shape,
    )(*args)


def kernel(x_gene, x_cell, x_gotem, edges_gene2cell, edges_cell2gene,
           edges_gene2gotem, edges_gotem2cell, Weg, beg, Wgo, bgo,
           W1, b1, W2, b2, W3, b3, Wd, bd):
    zeros = jnp.zeros((3136, CH), jnp.float32)
    ones_h = jnp.ones((GRP, DW), jnp.float32)
    zeros16 = jnp.zeros((3136, DW), jnp.float32)

    rels = [(edges_gene2cell, NG, NCN),
            (edges_cell2gene, NCN, NG),
            (edges_gene2gotem, NG, NGO),
            (edges_gotem2cell, NGO, NCN)]
    prep = [_prep_edges(e, ns_, nd_) for e, ns_, nd_ in rels]

    hist_in = []
    for (src_h, dst_h, _, _) in prep:
        hist_in += [src_h, dst_h]
    degs = _make_hist()(*hist_in, ones_h, zeros16)

    def _deg(i, n):
        return degs[i][:, :1]

    s0 = _deg(0, NG)
    d0 = _deg(1, NCN)
    s1 = _deg(2, NCN)
    d1 = _deg(3, NG)
    s2 = _deg(4, NG)
    d2 = _deg(5, NGO)
    s3 = _deg(6, NGO)
    d3 = _deg(7, NCN)

    def BATCH(convs):
        res = []
        for z, ri in convs:
            n_dst = rels[ri][2]
            f = _make_agg_batch((n_dst,))
            out, = f(z.reshape(-1, CH), prep[ri][2], prep[ri][3], zeros)
            res.append(out.reshape(n_dst, F))
        return res

    zx_g = _scale_call(x_gene, s0)
    zx_c = _scale_call(x_cell, s1)
    zx_go = _scale_call(x_gotem, s3)

    A_c2g, A_g2c, A_go2c = BATCH([(zx_c, 1), (zx_g, 0), (zx_go, 3)])
    g_s0, g_s2 = _post1(A_c2g, d1, Weg[1], beg[1], [s0, s2])
    c_s1, = _post2(A_g2c, d0, Weg[0], beg[0],
                   A_go2c, d3, Weg[3], beg[3], ns=[s1])

    A_g2go, A4, A6 = BATCH([(g_s2, 2), (g_s0, 0), (c_s1, 1)])
    go_s3, = _post1(A_g2go, d2, Wgo, bgo, [s3])
    hg1_s2, = _post1(A6, d1, W1[1], b1[1], [s2])

    A5, A8 = BATCH([(go_s3, 3), (hg1_s2, 2)])
    hc1_s1, = _post2(A4, d0, W1[0], b1[0], A5, d3, W1[3], b1[3], ns=[s1])
    hgo2_s3, = _post1(A8, d2, W2[2], b2[2], [s3])

    A7, A10 = BATCH([(hc1_s1, 1), (hgo2_s3, 3)])
    hg2_s0, = _post1(A7, d1, W2[1], b2[1], [s0])

    A9, = BATCH([(hg2_s0, 0)])
    out = _post2(A9, d0, W3[0], b3[0], A10, d3, W3[3], b3[3],
                 final=(Wd[1], bd[1]))
    return out


# speedup vs baseline: 2.3191x; 1.0342x over previous
import jax
import jax.numpy as jnp
from jax import lax
from jax.experimental import pallas as pl
from jax.experimental.pallas import tpu as pltpu
from jax.experimental.pallas import tpu_sc as plsc

NG = 50000
NCN = 50000
NGO = 10000
E = 150000
F = 128
CH = 32
NCHUNK = 4
NSUB = 16
GRP = 128
GPW = 74
EROWS = NSUB * GPW
EPAD = EROWS * GRP


def _acc_geom(n_dst):
    if n_dst == 50000:
        return 50176, 3136, 2960
    if n_dst == 10000:
        return 10240, 640, 400
    raise ValueError(n_dst)


def _conv_block(ztab, srcx, dstp, zeros, out, n_dst,
                src_v, dst_v, rows, rows2, acc, gsem, gsem2, c, s):
    ACC, STRIPE, LAST = _acc_geom(n_dst)
    pltpu.sync_copy(dstp.at[s], dst_v)
    for p in range(2):
        ck = 2 * p + c
        pltpu.sync_copy(srcx.at[ck, s], src_v)
        pltpu.sync_copy(zeros.at[pl.ds(0, STRIPE)],
                        acc.at[pl.ds(s * STRIPE, STRIPE)])
        plsc.subcore_barrier()

        def step(t, carry):
            g0 = 2 * t
            pltpu.async_copy(ztab.at[src_v.at[g0]], rows, gsem).wait()
            d1 = pltpu.async_copy(ztab.at[src_v.at[g0 + 1]], rows2, gsem2)
            pltpu.sync_copy(rows, acc.at[dst_v.at[g0]], add=True)
            d1.wait()
            pltpu.sync_copy(rows2, acc.at[dst_v.at[g0 + 1]], add=True)
            return carry

        lax.fori_loop(0, GPW // 2, step, 0)
        plsc.subcore_barrier()

        @pl.when(s < NSUB - 1)
        def _():
            pltpu.sync_copy(acc.at[pl.ds(s * STRIPE, STRIPE)],
                            out.at[pl.ds(s * STRIPE, STRIPE), ck])

        @pl.when(s == NSUB - 1)
        def _():
            pltpu.sync_copy(acc.at[pl.ds((NSUB - 1) * STRIPE, LAST)],
                            out.at[pl.ds((NSUB - 1) * STRIPE, LAST), ck])

        plsc.subcore_barrier()


def _make_agg_batch(n_dsts):
    k = len(n_dsts)
    mesh = plsc.VectorSubcoreMesh(core_axis_name="c", subcore_axis_name="s")

    def body(*refs):
        ztabs = refs[0:k]
        srcxs = refs[k:2 * k]
        dstps = refs[2 * k:3 * k]
        zeros = refs[3 * k]
        outs = refs[3 * k + 1:4 * k + 1]
        src_v, dst_v, rows, rows2, acc, gsem, gsem2 = refs[4 * k + 1:]
        c = lax.axis_index("c")
        s = lax.axis_index("s")
        for i in range(k):
            _conv_block(ztabs[i], srcxs[i], dstps[i], zeros, outs[i],
                        n_dsts[i], src_v, dst_v, rows, rows2, acc,
                        gsem, gsem2, c, s)

    return pl.kernel(
        body,
        out_type=[jax.ShapeDtypeStruct((n, NCHUNK, CH), jnp.float32)
                  for n in n_dsts],
        mesh=mesh,
        scratch_types=[
            pltpu.VMEM((GPW, GRP), jnp.int32),
            pltpu.VMEM((GPW, GRP), jnp.int32),
            pltpu.VMEM((GRP, CH), jnp.float32),
            pltpu.VMEM((GRP, CH), jnp.float32),
            pltpu.VMEM_SHARED((max(_acc_geom(n)[0] for n in n_dsts), CH),
                              jnp.float32),
            pltpu.SemaphoreType.DMA,
            pltpu.SemaphoreType.DMA,
        ],
        compiler_params=pltpu.CompilerParams(use_tc_tiling_on_sc=False),
    )


_HSIZES = [NG, NCN, NCN, NG, NG, NGO, NGO, NCN]
DW = 16


def _make_hist():
    mesh = plsc.VectorSubcoreMesh(core_axis_name="c", subcore_axis_name="s")
    out_types = [jax.ShapeDtypeStruct((n, DW), jnp.float32)
                 for n in _HSIZES]

    def body(*refs):
        idx_in = refs[0:8]
        ones_h, zeros16 = refs[8], refs[9]
        outs = refs[10:18]
        dst_v, ones_v, acc, hsem = refs[18:22]
        c = lax.axis_index("c")
        s = lax.axis_index("s")
        pltpu.sync_copy(ones_h, ones_v)
        for half in range(2):
            @pl.when(c == half)
            def _(half=half):
                for h in range(4 * half, 4 * half + 4):
                    n = _HSIZES[h]
                    _, STRIPE, LAST = _acc_geom(n)
                    pltpu.sync_copy(zeros16.at[pl.ds(0, STRIPE)],
                                    acc.at[pl.ds(s * STRIPE, STRIPE)])
                    pltpu.sync_copy(idx_in[h].at[s], dst_v)
                    plsc.subcore_barrier()

                    def grp(g, carry):
                        pltpu.sync_copy(ones_v, acc.at[dst_v.at[g]],
                                        add=True)
                        return carry

                    lax.fori_loop(0, GPW, grp, 0)
                    plsc.subcore_barrier()

                    @pl.when(s < NSUB - 1)
                    def _():
                        pltpu.sync_copy(
                            acc.at[pl.ds(s * STRIPE, STRIPE)],
                            outs[h].at[pl.ds(s * STRIPE, STRIPE)])

                    @pl.when(s == NSUB - 1)
                    def _():
                        pltpu.sync_copy(
                            acc.at[pl.ds((NSUB - 1) * STRIPE, LAST)],
                            outs[h].at[pl.ds((NSUB - 1) * STRIPE, LAST)])

                    plsc.subcore_barrier()

    return pl.kernel(
        body,
        out_type=out_types,
        mesh=mesh,
        scratch_types=[
            pltpu.VMEM((GPW, GRP), jnp.int32),
            pltpu.VMEM((GRP, DW), jnp.float32),
            pltpu.VMEM_SHARED((50176, DW), jnp.float32),
            pltpu.SemaphoreType.DMA,
        ],
        compiler_params=pltpu.CompilerParams(use_tc_tiling_on_sc=False),
    )


def _prep_edges(e, n_src, n_dst):
    src = e[0]
    dst = e[1]
    pad = EPAD - E
    src_h = jnp.concatenate(
        [src, jnp.full((pad,), n_src, jnp.int32)]).reshape(
            NSUB, GPW, GRP)
    dst_h = jnp.concatenate(
        [dst, jnp.full((pad,), n_dst, jnp.int32)]).reshape(
            NSUB, GPW, GRP)
    src0 = jnp.concatenate([src, jnp.zeros((pad,), jnp.int32)])
    srcx = (src0[None, :] * NCHUNK
            + jnp.arange(NCHUNK, dtype=jnp.int32)[:, None]).reshape(
                NCHUNK, NSUB, GPW, GRP)
    return src_h, dst_h, srcx, dst_h


_B = 2000


def _nrm(d):
    return lax.rsqrt(jnp.maximum(d, 1.0))


def _scale_call(x, deg):
    n = x.shape[0]

    def body(x_ref, d_ref, o_ref):
        o_ref[...] = x_ref[...] * _nrm(d_ref[...])

    return pl.pallas_call(
        body,
        grid=(n // _B,),
        in_specs=[pl.BlockSpec((_B, F), lambda i: (i, 0)),
                  pl.BlockSpec((_B, 1), lambda i: (i, 0))],
        out_specs=pl.BlockSpec((_B, F), lambda i: (i, 0)),
        out_shape=jax.ShapeDtypeStruct((n, F), jnp.float32),
    )(x, deg)


def _assemble(a_ref, d_ref):
    return a_ref[...] * _nrm(d_ref[...])


def _post1(agg, dd, W, b, ns_list):
    n = agg.shape[0]
    m = len(ns_list)

    def body(a_ref, d_ref, w_ref, b_ref, *rest):
        ns_refs = rest[:m]
        outs = rest[m:]
        xb = _assemble(a_ref, d_ref)
        y = jnp.maximum(
            jnp.dot(xb, w_ref[...], preferred_element_type=jnp.float32)
            + b_ref[...], 0.0)
        for j in range(m):
            outs[j][...] = y * _nrm(ns_refs[j][...])

    return pl.pallas_call(
        body,
        grid=(n // _B,),
        in_specs=[pl.BlockSpec((_B, F), lambda i: (i, 0)),
                  pl.BlockSpec((_B, 1), lambda i: (i, 0)),
                  pl.BlockSpec((F, F), lambda i: (0, 0)),
                  pl.BlockSpec((1, F), lambda i: (0, 0))]
        + [pl.BlockSpec((_B, 1), lambda i: (i, 0))] * m,
        out_specs=[pl.BlockSpec((_B, F), lambda i: (i, 0))] * m,
        out_shape=[jax.ShapeDtypeStruct((n, F), jnp.float32)] * m,
    )(agg, dd, W, b.reshape(1, F), *ns_list)


def _post2(agg1, dd1, Wa, ba, agg2, dd2, Wb, bb, ns=None, final=None):
    n = agg1.shape[0]
    m = len(ns) if ns is not None else 0

    def body(a1, d1, wa, ba_, a2, d2, wb, bb_, *rest):
        y = (jnp.maximum(
                jnp.dot(_assemble(a1, d1), wa[...],
                        preferred_element_type=jnp.float32) + ba_[...], 0.0)
             + jnp.maximum(
                jnp.dot(_assemble(a2, d2), wb[...],
                        preferred_element_type=jnp.float32) + bb_[...], 0.0))
        if final is not None:
            wd, bd_, out = rest
            out[...] = (jnp.dot(y, wd[...],
                                preferred_element_type=jnp.float32)
                        + bd_[...])
        else:
            ns_refs = rest[:m]
            outs = rest[m:]
            for j in range(m):
                outs[j][...] = y * _nrm(ns_refs[j][...])

    base_specs = [pl.BlockSpec((_B, F), lambda i: (i, 0)),
                  pl.BlockSpec((_B, 1), lambda i: (i, 0)),
                  pl.BlockSpec((F, F), lambda i: (0, 0)),
                  pl.BlockSpec((1, F), lambda i: (0, 0))]
    in_specs = base_specs + base_specs
    args = [agg1, dd1, Wa, ba.reshape(1, F), agg2, dd2, Wb, bb.reshape(1, F)]
    if final is not None:
        wd, bd_ = final
        in_specs += [pl.BlockSpec((F, F), lambda i: (0, 0)),
                     pl.BlockSpec((1, F), lambda i: (0, 0))]
        args += [wd, bd_.reshape(1, F)]
        out_specs = pl.BlockSpec((_B, F), lambda i: (i, 0))
        out_shape = jax.ShapeDtypeStruct((n, F), jnp.float32)
    else:
        in_specs += [pl.BlockSpec((_B, 1), lambda i: (i, 0))] * m
        args += list(ns)
        out_specs = [pl.BlockSpec((_B, F), lambda i: (i, 0))] * m
        out_shape = [jax.ShapeDtypeStruct((n, F), jnp.float32)] * m
    return pl.pallas_call(
        body,
        grid=(n // _B,),
        in_specs=in_specs,
        out_specs=out_specs,
        out_shape=out_shape,
    )(*args)


def kernel(x_gene, x_cell, x_gotem, edges_gene2cell, edges_cell2gene,
           edges_gene2gotem, edges_gotem2cell, Weg, beg, Wgo, bgo,
           W1, b1, W2, b2, W3, b3, Wd, bd):
    zeros = jnp.zeros((3136, CH), jnp.float32)
    ones_h = jnp.ones((GRP, DW), jnp.float32)
    zeros16 = jnp.zeros((3136, DW), jnp.float32)

    rels = [(edges_gene2cell, NG, NCN),
            (edges_cell2gene, NCN, NG),
            (edges_gene2gotem, NG, NGO),
            (edges_gotem2cell, NGO, NCN)]
    prep = [_prep_edges(e, ns_, nd_) for e, ns_, nd_ in rels]

    hist_in = []
    for (src_h, dst_h, _, _) in prep:
        hist_in += [src_h, dst_h]
    degs = _make_hist()(*hist_in, ones_h, zeros16)

    def _deg(i, n):
        return degs[i][:, :1]

    s0 = _deg(0, NG)
    d0 = _deg(1, NCN)
    s1 = _deg(2, NCN)
    d1 = _deg(3, NG)
    s2 = _deg(4, NG)
    d2 = _deg(5, NGO)
    s3 = _deg(6, NGO)
    d3 = _deg(7, NCN)

    def BATCH(convs):
        res = []
        for z, ri in convs:
            n_dst = rels[ri][2]
            f = _make_agg_batch((n_dst,))
            out, = f(z.reshape(-1, CH), prep[ri][2], prep[ri][3], zeros)
            res.append(out.reshape(n_dst, F))
        return res

    zx_g = _scale_call(x_gene, s0)
    zx_c = _scale_call(x_cell, s1)
    zx_go = _scale_call(x_gotem, s3)

    A_c2g, A_g2c, A_go2c = BATCH([(zx_c, 1), (zx_g, 0), (zx_go, 3)])
    g_s0, g_s2 = _post1(A_c2g, d1, Weg[1], beg[1], [s0, s2])
    c_s1, = _post2(A_g2c, d0, Weg[0], beg[0],
                   A_go2c, d3, Weg[3], beg[3], ns=[s1])

    A_g2go, A4, A6 = BATCH([(g_s2, 2), (g_s0, 0), (c_s1, 1)])
    go_s3, = _post1(A_g2go, d2, Wgo, bgo, [s3])
    hg1_s2, = _post1(A6, d1, W1[1], b1[1], [s2])

    A5, A8 = BATCH([(go_s3, 3), (hg1_s2, 2)])
    hc1_s1, = _post2(A4, d0, W1[0], b1[0], A5, d3, W1[3], b1[3], ns=[s1])
    hgo2_s3, = _post1(A8, d2, W2[2], b2[2], [s3])

    A7, A10 = BATCH([(hc1_s1, 1), (hgo2_s3, 3)])
    hg2_s0, = _post1(A7, d1, W2[1], b2[1], [s0])

    A9, = BATCH([(hg2_s0, 0)])
    out = _post2(A9, d0, W3[0], b3[0], A10, d3, W3[3], b3[3],
                 final=(Wd[1], bd[1]))
    return out
